# Initial kernel scaffold; baseline (speedup 1.0000x reference)
import jax
import jax.numpy as jnp
from jax.experimental import pallas as pl


def kernel(x, pos_embedding):
    raise NotImplementedError("write your pallas kernel here")


# speedup vs baseline: 1.7548x; 1.7548x over previous
import functools

import jax
import jax.numpy as jnp
from jax import lax
from jax.experimental import pallas as pl
from jax.experimental.pallas import tpu as pltpu
from jax.experimental.pallas import tpu_sc as plsc


def kernel(x, pos_embedding):
    S, N = x.shape
    _, D = pos_embedding.shape

    info = plsc.get_sparse_core_info()
    NW = info.num_cores * info.num_subcores
    rows_per_w = S // NW
    BS = 64
    n_chunks = rows_per_w // BS

    mesh = plsc.VectorSubcoreMesh(core_axis_name="c", subcore_axis_name="s")

    @functools.partial(
        pl.kernel,
        out_type=jax.ShapeDtypeStruct((S, N * D), jnp.float32),
        mesh=mesh,
        scratch_types=[pltpu.VMEM((BS, D), jnp.float32)],
    )
    def body(table_hbm, out_hbm, buf):
        wid = lax.axis_index("s") * info.num_cores + lax.axis_index("c")
        base0 = wid * rows_per_w
        for c in range(n_chunks):
            base = base0 + c * BS
            pltpu.sync_copy(table_hbm.at[pl.ds(base, BS)], buf)
            for n in range(N):
                pltpu.sync_copy(buf, out_hbm.at[pl.ds(base, BS), pl.ds(n * D, D)])

    return body(pos_embedding).reshape(S, N, D)
